# Initial kernel scaffold; baseline (speedup 1.0000x reference)
import jax
import jax.numpy as jnp
from jax.experimental import pallas as pl


def kernel(box_prediction, class_prediction):
    raise NotImplementedError("write your pallas kernel here")


# speedup vs baseline: 1.3823x; 1.3823x over previous
import jax
import jax.numpy as jnp
from jax.experimental import pallas as pl
from jax.experimental.pallas import tpu as pltpu

_B, _N, _C = 4, 20000, 80
_MAX_DET = 100
_IOU_THR = 0.5
_SCORE_THR = 0.5
_NEG = -1e30
_TILE = 2000
_NT = _N // _TILE


def _nms_kernel(box_ref, cls_ref, nms_box_ref, nms_cls_ref, cls_out_ref,
                soft_s, scores_s, ymin_s, xmin_s, ymax_s, xmax_s, area_s):
    t = pl.program_id(1)

    z = jnp.square(cls_ref[0] * 10.0)
    z = z - jnp.max(z, axis=1, keepdims=True)
    e = jnp.exp(z)
    p = e / jnp.sum(e, axis=1, keepdims=True)
    cls_out_ref[0] = p
    soft_s[pl.ds(t * _TILE, _TILE), :] = p

    s = jnp.max(p, axis=1, keepdims=True)
    sw = jnp.where(s >= _SCORE_THR, s, _NEG)
    scores_s[pl.ds(t, 1), :] = sw.reshape(1, _TILE)

    bt = box_ref[0, pl.ds(t * _TILE, _TILE), :]
    y0, x0, y1, x1 = bt[:, 0:1], bt[:, 1:2], bt[:, 2:3], bt[:, 3:4]
    ymin = jnp.minimum(y0, y1)
    xmin = jnp.minimum(x0, x1)
    ymax = jnp.maximum(y0, y1)
    xmax = jnp.maximum(x0, x1)
    area = (ymax - ymin) * (xmax - xmin)
    ymin_s[pl.ds(t, 1), :] = ymin.reshape(1, _TILE)
    xmin_s[pl.ds(t, 1), :] = xmin.reshape(1, _TILE)
    ymax_s[pl.ds(t, 1), :] = ymax.reshape(1, _TILE)
    xmax_s[pl.ds(t, 1), :] = xmax.reshape(1, _TILE)
    area_s[pl.ds(t, 1), :] = area.reshape(1, _TILE)

    @pl.when(t == _NT - 1)
    def _():
        rows = jax.lax.broadcasted_iota(jnp.int32, (_NT, _TILE), 0)
        cols = jax.lax.broadcasted_iota(jnp.int32, (_NT, _TILE), 1)
        n_iota = rows * _TILE + cols
        lane_iota = jax.lax.broadcasted_iota(jnp.int32, (1, _TILE), 1)

        def body(i, _):
            scores_w = scores_s[...]
            m = jnp.max(scores_w)
            valid = m > (_NEG * 0.5)
            n_best = jnp.min(jnp.where(scores_w == m, n_iota, jnp.int32(2 ** 30)))
            r_i = n_best // _TILE
            c_one = lane_iota == (n_best % _TILE)

            def pick(ref):
                return jnp.sum(jnp.where(c_one, ref[pl.ds(r_i, 1), :], 0.0))

            ymin_b = pick(ymin_s)
            xmin_b = pick(xmin_s)
            ymax_b = pick(ymax_s)
            xmax_b = pick(xmax_s)
            area_b = pick(area_s)

            inter_h = jnp.maximum(
                0.0, jnp.minimum(ymax_b, ymax_s[...]) - jnp.maximum(ymin_b, ymin_s[...]))
            inter_w = jnp.maximum(
                0.0, jnp.minimum(xmax_b, xmax_s[...]) - jnp.maximum(xmin_b, xmin_s[...]))
            inter = inter_h * inter_w
            union = area_b + area_s[...] - inter
            iou = jnp.where(union > 0.0, inter / union, 0.0)
            suppress = (iou > _IOU_THR) | (n_iota == n_best)
            scores_s[...] = jnp.where(
                valid, jnp.where(suppress, _NEG, scores_w), scores_w)

            vf = jnp.where(valid, 1.0, 0.0)
            nms_box_ref[0, pl.ds(i, 1), :] = box_ref[0, pl.ds(n_best, 1), :] * vf
            nms_cls_ref[0, pl.ds(i, 1), :] = soft_s[pl.ds(n_best, 1), :] * vf
            return 0

        jax.lax.fori_loop(0, _MAX_DET, body, 0)


def kernel(box_prediction, class_prediction):
    grid = (_B, _NT)
    out = pl.pallas_call(
        _nms_kernel,
        grid=grid,
        in_specs=[
            pl.BlockSpec((1, _N, 4), lambda b, t: (b, 0, 0)),
            pl.BlockSpec((1, _TILE, _C), lambda b, t: (b, t, 0)),
        ],
        out_specs=[
            pl.BlockSpec((1, _MAX_DET, 4), lambda b, t: (b, 0, 0)),
            pl.BlockSpec((1, _MAX_DET, _C), lambda b, t: (b, 0, 0)),
            pl.BlockSpec((1, _TILE, _C), lambda b, t: (b, t, 0)),
        ],
        out_shape=[
            jax.ShapeDtypeStruct((_B, _MAX_DET, 4), jnp.float32),
            jax.ShapeDtypeStruct((_B, _MAX_DET, _C), jnp.float32),
            jax.ShapeDtypeStruct((_B, _N, _C), jnp.float32),
        ],
        scratch_shapes=[
            pltpu.VMEM((_N, _C), jnp.float32),
            pltpu.VMEM((_NT, _TILE), jnp.float32),
            pltpu.VMEM((_NT, _TILE), jnp.float32),
            pltpu.VMEM((_NT, _TILE), jnp.float32),
            pltpu.VMEM((_NT, _TILE), jnp.float32),
            pltpu.VMEM((_NT, _TILE), jnp.float32),
            pltpu.VMEM((_NT, _TILE), jnp.float32),
        ],
    )(box_prediction, class_prediction)
    nms_box, nms_cls, cls_predictions = out
    return nms_box, nms_cls, cls_predictions
